# Initial kernel scaffold; baseline (speedup 1.0000x reference)
import jax
import jax.numpy as jnp
from jax.experimental import pallas as pl


def kernel(disc0_lookup, disc0_table, disc1_lookup, disc1_table, disc2_lookup, disc2_table, disc3_lookup, disc3_table, cont0_values, cont0_indicators, cont0_linear_w, cont0_linear_b, cont0_pos_table, ent0_lookup, ent0_table):
    raise NotImplementedError("write your pallas kernel here")


# speedup vs baseline: 1.4129x; 1.4129x over previous
import functools
import math

import jax
import jax.numpy as jnp
import numpy as np
from jax import lax
from jax.experimental import pallas as pl
from jax.experimental.pallas import tpu as pltpu
from jax.experimental.pallas import tpu_sc as plsc

N, L, D = 4096, 20, 64
PREC = 8
C = 6
B = N * L

NC, NS = 2, 16
NW = NC * NS
BPW = B // NW
CHUNK = 128
NCHUNK = BPW // CHUNK

_FW = (np.logspace(-PREC, 1, PREC, base=2.0) * math.pi).astype(np.float32)


_CONT_BLK = 2048


def _cont_body(v_ref, fw_ref, ind_ref, wt_ref, b_ref, pos_ref, o_ref):
    v = v_ref[...]
    w = v * fw_ref[...]
    f = jnp.concatenate([jnp.sin(w), jnp.cos(w)], axis=1)
    proj = jnp.dot(f, wt_ref[...], preferred_element_type=jnp.float32)
    onehot = (ind_ref[...] == lax.broadcasted_iota(jnp.int32, (1, 4), 1)
              ).astype(jnp.float32)
    pos = jnp.dot(onehot, pos_ref[...], preferred_element_type=jnp.float32)
    o_ref[...] = proj + b_ref[...] + pos


def _cont_embed(values, indicators, linear_w, linear_b, pos_table):
    vals = values.reshape(B, 1)
    inds = indicators.reshape(B, 1).astype(jnp.int32)
    wt = linear_w.T
    bias = linear_b.reshape(1, D)
    grid = (B // _CONT_BLK,)
    return pl.pallas_call(
        _cont_body,
        grid=grid,
        in_specs=[
            pl.BlockSpec((_CONT_BLK, 1), lambda i: (i, 0)),
            pl.BlockSpec((1, PREC), lambda i: (0, 0)),
            pl.BlockSpec((_CONT_BLK, 1), lambda i: (i, 0)),
            pl.BlockSpec((2 * PREC, D), lambda i: (0, 0)),
            pl.BlockSpec((1, D), lambda i: (0, 0)),
            pl.BlockSpec((4, D), lambda i: (0, 0)),
        ],
        out_specs=pl.BlockSpec((_CONT_BLK, D), lambda i: (i, 0)),
        out_shape=jax.ShapeDtypeStruct((B, D), jnp.float32),
    )(vals, jnp.asarray(_FW).reshape(1, PREC), inds, wt, bias, pos_table)


def _sc_body(d0t, d1t, d2t, d3t, entt, cont,
             i0, i1, i2, i3, ie,
             out,
             x0, x1, x2, x3, xe,
             r0, r1, r2, r3, re, rc, sem):
    wid = lax.axis_index("s") * NC + lax.axis_index("c")
    base = wid * BPW

    @pl.loop(0, NCHUNK)
    def _chunk(k):
        off = base + k * CHUNK
        sl = pl.ds(off, CHUNK)
        pltpu.sync_copy(i0.at[sl], x0)
        pltpu.sync_copy(i1.at[sl], x1)
        pltpu.sync_copy(i2.at[sl], x2)
        pltpu.sync_copy(i3.at[sl], x3)
        pltpu.sync_copy(ie.at[sl], xe)
        g0 = pltpu.async_copy(d0t.at[x0], r0, sem)
        g1 = pltpu.async_copy(d1t.at[x1], r1, sem)
        g2 = pltpu.async_copy(d2t.at[x2], r2, sem)
        g3 = pltpu.async_copy(d3t.at[x3], r3, sem)
        ge = pltpu.async_copy(entt.at[xe], re, sem)
        gc = pltpu.async_copy(cont.at[sl], rc, sem)
        g0.wait()
        g1.wait()
        g2.wait()
        g3.wait()
        ge.wait()
        gc.wait()
        w0 = pltpu.async_copy(r0, out.at[sl, 0], sem)
        w1 = pltpu.async_copy(r1, out.at[sl, 1], sem)
        w2 = pltpu.async_copy(r2, out.at[sl, 2], sem)
        w3 = pltpu.async_copy(r3, out.at[sl, 3], sem)
        wc = pltpu.async_copy(rc, out.at[sl, 4], sem)
        we = pltpu.async_copy(re, out.at[sl, 5], sem)
        w0.wait()
        w1.wait()
        w2.wait()
        w3.wait()
        wc.wait()
        we.wait()


def _sc_gather(d0t, d1t, d2t, d3t, entt, cont, i0, i1, i2, i3, ie):
    mesh = plsc.VectorSubcoreMesh(
        core_axis_name="c", subcore_axis_name="s",
        num_cores=NC, num_subcores=NS)
    f = pl.kernel(
        _sc_body,
        out_type=jax.ShapeDtypeStruct((B, C, D), jnp.float32),
        mesh=mesh,
        scratch_types=(
            [pltpu.VMEM((CHUNK,), jnp.int32) for _ in range(5)]
            + [pltpu.VMEM((CHUNK, D), jnp.float32) for _ in range(6)]
            + [pltpu.SemaphoreType.DMA]
        ),
        compiler_params=pltpu.CompilerParams(use_tc_tiling_on_sc=False),
    )
    return f(d0t, d1t, d2t, d3t, entt, cont, i0, i1, i2, i3, ie)


def kernel(disc0_lookup, disc0_table, disc1_lookup, disc1_table,
           disc2_lookup, disc2_table, disc3_lookup, disc3_table,
           cont0_values, cont0_indicators, cont0_linear_w, cont0_linear_b,
           cont0_pos_table, ent0_lookup, ent0_table):
    cont_emb = _cont_embed(cont0_values, cont0_indicators,
                           cont0_linear_w, cont0_linear_b, cont0_pos_table)
    i0 = disc0_lookup.reshape(B).astype(jnp.int32)
    i1 = disc1_lookup.reshape(B).astype(jnp.int32)
    i2 = disc2_lookup.reshape(B).astype(jnp.int32)
    i3 = disc3_lookup.reshape(B).astype(jnp.int32)
    ie = ent0_lookup.reshape(B).astype(jnp.int32)
    out = _sc_gather(disc0_table, disc1_table, disc2_table, disc3_table,
                     ent0_table, cont_emb, i0, i1, i2, i3, ie)
    return out.reshape(N, L, C, D)


# speedup vs baseline: 2.9071x; 2.0575x over previous
import math

import jax
import jax.numpy as jnp
import numpy as np
from jax import lax
from jax.experimental import pallas as pl
from jax.experimental.pallas import tpu as pltpu
from jax.experimental.pallas import tpu_sc as plsc

N, L, D = 4096, 20, 64
PREC = 8
C = 6
B = N * L
NENT = 204
H = N // 2

NC, NS = 2, 16
NW = NC * NS
BPW = B // NW
CHUNK = 512
SUB = 128
NSUB = CHUNK // SUB
NCHUNK = BPW // CHUNK

_FW = (np.logspace(-PREC, 1, PREC, base=2.0) * math.pi).astype(np.float32)


def _permute(lookup):
    lt = lookup.T.astype(jnp.int32)
    return lt.reshape(L, 2, H).transpose(0, 2, 1).reshape(B)


def _sc_body(d0t, d1t, d2t, d3t, i0, i1, i2, i3, out,
             xa, xb, ra, rb, sem, semw):
    wid = lax.axis_index("s") * NC + lax.axis_index("c")
    base = wid * BPW
    tables = (d0t, d1t, d2t, d3t)
    idxs = (i0, i1, i2, i3)
    chunks = [(f, k) for f in range(4) for k in range(NCHUNK)]

    gathers = []
    writebacks = []

    def fire(i):
        f, k = chunks[i]
        xv, rv = (xa, ra) if i % 2 == 0 else (xb, rb)
        off = base + k * CHUNK
        pltpu.sync_copy(idxs[f].at[pl.ds(off, CHUNK)], xv)
        gs = []
        for j in range(NSUB):
            gs.append(pltpu.async_copy(
                tables[f].at[xv.at[pl.ds(j * SUB, SUB)]],
                rv.at[pl.ds(j * SUB, SUB)], sem))
        gathers.append(gs)

    def retire(i):
        f, k = chunks[i]
        rv = ra if i % 2 == 0 else rb
        off = base + k * CHUNK
        for g in gathers[i]:
            g.wait()
        writebacks.append(
            pltpu.async_copy(rv, out.at[f, pl.ds(off, CHUNK)], semw))

    n = len(chunks)
    fire(0)
    for i in range(1, n):
        if i >= 2:
            writebacks[i - 2].wait()
        fire(i)
        retire(i - 1)
    retire(n - 1)
    writebacks[n - 2].wait()
    writebacks[n - 1].wait()


def _sc_gather(d0t, d1t, d2t, d3t, i0, i1, i2, i3):
    mesh = plsc.VectorSubcoreMesh(
        core_axis_name="c", subcore_axis_name="s",
        num_cores=NC, num_subcores=NS)
    f = pl.kernel(
        _sc_body,
        out_type=jax.ShapeDtypeStruct((4, B, D), jnp.float32),
        mesh=mesh,
        scratch_types=[
            pltpu.VMEM((CHUNK,), jnp.int32),
            pltpu.VMEM((CHUNK,), jnp.int32),
            pltpu.VMEM((CHUNK, D), jnp.float32),
            pltpu.VMEM((CHUNK, D), jnp.float32),
            pltpu.SemaphoreType.DMA,
            pltpu.SemaphoreType.DMA,
        ],
        compiler_params=pltpu.CompilerParams(use_tc_tiling_on_sc=False),
    )
    return f(d0t, d1t, d2t, d3t, i0, i1, i2, i3)


def _tc_body(g_ref, v_ref, ind_ref, eid_ref, fw_ref, wt_ref, b_ref,
             pos_ref, ent_ref, o_ref):
    for f in range(4):
        m = g_ref[f]
        a = jnp.transpose(m[:, :D])
        b = jnp.transpose(m[:, D:])
        o_ref[0, f] = jnp.concatenate([a, b], axis=1)

    v = v_ref[0]
    w = fw_ref[...] * v
    feats = jnp.concatenate([jnp.sin(w), jnp.cos(w)], axis=0)
    proj = lax.dot_general(wt_ref[...], feats, (((0,), (0,)), ((), ())),
                           preferred_element_type=jnp.float32)
    ind = ind_ref[0]
    oh = (lax.broadcasted_iota(jnp.int32, (4, N), 0) == ind
          ).astype(jnp.float32)
    posc = jnp.dot(pos_ref[...], oh,
                   preferred_element_type=jnp.float32)
    o_ref[0, 4] = proj + b_ref[...] + posc

    eid = eid_ref[0]
    ohe = (lax.broadcasted_iota(jnp.int32, (NENT, N), 0) == eid
           ).astype(jnp.float32)
    o_ref[0, 5] = jnp.dot(ent_ref[...], ohe,
                          preferred_element_type=jnp.float32)


def _tc_assemble(g, values, indicators, ent_idx, linear_w, linear_b,
                 pos_table, ent_table):
    gr = g.reshape(4, B // 2, 128)
    vt = values.T.reshape(L, 1, N)
    it = indicators.T.astype(jnp.int32).reshape(L, 1, N)
    et = ent_idx.T.astype(jnp.int32).reshape(L, 1, N)
    fwc = jnp.asarray(_FW).reshape(PREC, 1)
    wt = linear_w.T
    bias = linear_b.reshape(D, 1)
    post = pos_table.T
    entt = ent_table.T
    out = pl.pallas_call(
        _tc_body,
        grid=(L,),
        in_specs=[
            pl.BlockSpec((4, H, 128), lambda l: (0, l, 0)),
            pl.BlockSpec((1, 1, N), lambda l: (l, 0, 0)),
            pl.BlockSpec((1, 1, N), lambda l: (l, 0, 0)),
            pl.BlockSpec((1, 1, N), lambda l: (l, 0, 0)),
            pl.BlockSpec((PREC, 1), lambda l: (0, 0)),
            pl.BlockSpec((2 * PREC, D), lambda l: (0, 0)),
            pl.BlockSpec((D, 1), lambda l: (0, 0)),
            pl.BlockSpec((D, 4), lambda l: (0, 0)),
            pl.BlockSpec((D, NENT), lambda l: (0, 0)),
        ],
        out_specs=pl.BlockSpec((1, C, D, N), lambda l: (l, 0, 0, 0)),
        out_shape=jax.ShapeDtypeStruct((L, C, D, N), jnp.float32),
    )(gr, vt, it, et, fwc, wt, bias, post, entt)
    return out


def kernel(disc0_lookup, disc0_table, disc1_lookup, disc1_table,
           disc2_lookup, disc2_table, disc3_lookup, disc3_table,
           cont0_values, cont0_indicators, cont0_linear_w, cont0_linear_b,
           cont0_pos_table, ent0_lookup, ent0_table):
    i0 = _permute(disc0_lookup)
    i1 = _permute(disc1_lookup)
    i2 = _permute(disc2_lookup)
    i3 = _permute(disc3_lookup)
    g = _sc_gather(disc0_table, disc1_table, disc2_table, disc3_table,
                   i0, i1, i2, i3)
    out = _tc_assemble(g, cont0_values, cont0_indicators, ent0_lookup,
                       cont0_linear_w, cont0_linear_b, cont0_pos_table,
                       ent0_table)
    return out.transpose(3, 0, 1, 2)


# speedup vs baseline: 3.5616x; 1.2252x over previous
import math

import jax
import jax.numpy as jnp
import numpy as np
from jax import lax
from jax.experimental import pallas as pl
from jax.experimental.pallas import tpu as pltpu
from jax.experimental.pallas import tpu_sc as plsc

N, L, D = 4096, 20, 64
PREC = 8
C = 6
B = N * L
NENT = 204
H = N // 2

NC, NS = 2, 16
NW = NC * NS
BPW = B // NW
CHUNK = 512
SUB = 128
NSUB = CHUNK // SUB
NCHUNK = BPW // CHUNK

_FW = (np.logspace(-PREC, 1, PREC, base=2.0) * math.pi).astype(np.float32)


HC = CHUNK // 2


def _sc_body(d0t, d1t, d2t, d3t, i0, i1, i2, i3, out,
             xa, xb, ra, rb, sem, semw):
    wid = lax.axis_index("s") * NC + lax.axis_index("c")
    base = wid * BPW
    tables = (d0t, d1t, d2t, d3t)
    idxs = (i0, i1, i2, i3)
    chunks = [(f, k) for f in range(4) for k in range(NCHUNK)]

    gathers = []
    writebacks = []

    def fire(i):
        f, k = chunks[i]
        xv, rv = (xa, ra) if i % 2 == 0 else (xb, rb)
        off = base + k * CHUNK
        l = off // N
        a0 = pl.multiple_of((off % N) // 2, HC)
        pltpu.sync_copy(idxs[f].at[l, pl.ds(a0, HC)], xv.at[pl.ds(0, HC)])
        pltpu.sync_copy(idxs[f].at[l, pl.ds(H + a0, HC)],
                        xv.at[pl.ds(HC, HC)])
        gs = []
        for j in range(NSUB):
            gs.append(pltpu.async_copy(
                tables[f].at[xv.at[pl.ds(j * SUB, SUB)]],
                rv.at[pl.ds(j * SUB, SUB)], sem))
        gathers.append(gs)

    def retire(i):
        f, k = chunks[i]
        rv = ra if i % 2 == 0 else rb
        off = base + k * CHUNK
        p0 = pl.multiple_of(off // 2, HC)
        for g in gathers[i]:
            g.wait()
        writebacks.append(pltpu.async_copy(
            rv.at[pl.ds(0, HC)],
            out.at[f, pl.ds(p0, HC), pl.ds(0, D)], semw))
        writebacks.append(pltpu.async_copy(
            rv.at[pl.ds(HC, HC)],
            out.at[f, pl.ds(p0, HC), pl.ds(D, D)], semw))

    n = len(chunks)
    fire(0)
    for i in range(1, n):
        if i >= 2:
            writebacks[2 * (i - 2)].wait()
            writebacks[2 * (i - 2) + 1].wait()
        fire(i)
        retire(i - 1)
    retire(n - 1)
    for wb in writebacks[2 * (n - 2):]:
        wb.wait()


def _sc_gather(d0t, d1t, d2t, d3t, i0, i1, i2, i3):
    mesh = plsc.VectorSubcoreMesh(
        core_axis_name="c", subcore_axis_name="s",
        num_cores=NC, num_subcores=NS)
    f = pl.kernel(
        _sc_body,
        out_type=jax.ShapeDtypeStruct((4, B // 2, 2 * D), jnp.float32),
        mesh=mesh,
        scratch_types=[
            pltpu.VMEM((CHUNK,), jnp.int32),
            pltpu.VMEM((CHUNK,), jnp.int32),
            pltpu.VMEM((CHUNK, D), jnp.float32),
            pltpu.VMEM((CHUNK, D), jnp.float32),
            pltpu.SemaphoreType.DMA,
            pltpu.SemaphoreType.DMA,
        ],
        compiler_params=pltpu.CompilerParams(use_tc_tiling_on_sc=False),
    )
    return f(d0t, d1t, d2t, d3t, i0, i1, i2, i3)


def _tc_body(g_ref, v_ref, ind_ref, eid_ref, fw_ref, wt_ref, b_ref,
             pos_ref, ent_ref, o_ref):
    for f in range(4):
        m = g_ref[f]
        a = jnp.transpose(m[:, :D])
        b = jnp.transpose(m[:, D:])
        o_ref[0, f] = jnp.concatenate([a, b], axis=1)

    v = v_ref[0]
    w = fw_ref[...] * v
    feats = jnp.concatenate([jnp.sin(w), jnp.cos(w)], axis=0)
    proj = lax.dot_general(wt_ref[...], feats, (((0,), (0,)), ((), ())),
                           preferred_element_type=jnp.float32)
    ind = ind_ref[0]
    oh = (lax.broadcasted_iota(jnp.int32, (4, N), 0) == ind
          ).astype(jnp.float32)
    posc = jnp.dot(pos_ref[...], oh,
                   preferred_element_type=jnp.float32)
    o_ref[0, 4] = proj + b_ref[...] + posc

    eid = eid_ref[0]
    ohe = (lax.broadcasted_iota(jnp.int32, (NENT, N), 0) == eid
           ).astype(jnp.float32)
    o_ref[0, 5] = jnp.dot(ent_ref[...], ohe,
                          preferred_element_type=jnp.float32)


def _tc_assemble(g, values, indicators, ent_idx, linear_w, linear_b,
                 pos_table, ent_table):
    gr = g.reshape(4, B // 2, 128)
    vt = values.T.reshape(L, 1, N)
    it = indicators.T.astype(jnp.int32).reshape(L, 1, N)
    et = ent_idx.T.astype(jnp.int32).reshape(L, 1, N)
    fwc = jnp.asarray(_FW).reshape(PREC, 1)
    wt = linear_w.T
    bias = linear_b.reshape(D, 1)
    post = pos_table.T
    entt = ent_table.T
    out = pl.pallas_call(
        _tc_body,
        grid=(L,),
        in_specs=[
            pl.BlockSpec((4, H, 128), lambda l: (0, l, 0)),
            pl.BlockSpec((1, 1, N), lambda l: (l, 0, 0)),
            pl.BlockSpec((1, 1, N), lambda l: (l, 0, 0)),
            pl.BlockSpec((1, 1, N), lambda l: (l, 0, 0)),
            pl.BlockSpec((PREC, 1), lambda l: (0, 0)),
            pl.BlockSpec((2 * PREC, D), lambda l: (0, 0)),
            pl.BlockSpec((D, 1), lambda l: (0, 0)),
            pl.BlockSpec((D, 4), lambda l: (0, 0)),
            pl.BlockSpec((D, NENT), lambda l: (0, 0)),
        ],
        out_specs=pl.BlockSpec((1, C, D, N), lambda l: (l, 0, 0, 0)),
        out_shape=jax.ShapeDtypeStruct((L, C, D, N), jnp.float32),
    )(gr, vt, it, et, fwc, wt, bias, post, entt)
    return out


def kernel(disc0_lookup, disc0_table, disc1_lookup, disc1_table,
           disc2_lookup, disc2_table, disc3_lookup, disc3_table,
           cont0_values, cont0_indicators, cont0_linear_w, cont0_linear_b,
           cont0_pos_table, ent0_lookup, ent0_table):
    i0 = disc0_lookup.T.astype(jnp.int32)
    i1 = disc1_lookup.T.astype(jnp.int32)
    i2 = disc2_lookup.T.astype(jnp.int32)
    i3 = disc3_lookup.T.astype(jnp.int32)
    g = _sc_gather(disc0_table, disc1_table, disc2_table, disc3_table,
                   i0, i1, i2, i3)
    out = _tc_assemble(g, cont0_values, cont0_indicators, ent0_lookup,
                       cont0_linear_w, cont0_linear_b, cont0_pos_table,
                       ent0_table)
    return out.transpose(3, 0, 1, 2)


# speedup vs baseline: 3.7566x; 1.0548x over previous
import math

import jax
import jax.numpy as jnp
import numpy as np
from jax import lax
from jax.experimental import pallas as pl
from jax.experimental.pallas import tpu as pltpu
from jax.experimental.pallas import tpu_sc as plsc

N, L, D = 4096, 20, 64
PREC = 8
C = 6
B = N * L
NENT = 204
H = N // 2

NC, NS = 2, 16
NW = NC * NS
BPW = B // NW
CHUNK = 512
SUB = 128
NSUB = CHUNK // SUB
NCHUNK = BPW // CHUNK

_FW = (np.logspace(-PREC, 1, PREC, base=2.0) * math.pi).astype(np.float32)


HC = CHUNK // 2


def _sc_body(table, idx, out, xa, xb, ra, rb, sem, semw):
    wid = lax.axis_index("s") * NC + lax.axis_index("c")
    base = wid * BPW

    gathers = []
    writebacks = []

    def fire(i):
        xv, rv = (xa, ra) if i % 2 == 0 else (xb, rb)
        off = base + i * CHUNK
        l = off // N
        a0 = pl.multiple_of((off % N) // 2, HC)
        pltpu.sync_copy(idx.at[l, pl.ds(a0, HC)], xv.at[pl.ds(0, HC)])
        pltpu.sync_copy(idx.at[l, pl.ds(H + a0, HC)], xv.at[pl.ds(HC, HC)])
        gs = []
        for j in range(NSUB):
            gs.append(pltpu.async_copy(
                table.at[xv.at[pl.ds(j * SUB, SUB)]],
                rv.at[pl.ds(j * SUB, SUB)], sem))
        gathers.append(gs)

    def retire(i):
        rv = ra if i % 2 == 0 else rb
        off = base + i * CHUNK
        p0 = pl.multiple_of(off // 2, HC)
        for g in gathers[i]:
            g.wait()
        writebacks.append(pltpu.async_copy(
            rv.at[pl.ds(0, HC)],
            out.at[pl.ds(p0, HC), pl.ds(0, D)], semw))
        writebacks.append(pltpu.async_copy(
            rv.at[pl.ds(HC, HC)],
            out.at[pl.ds(p0, HC), pl.ds(D, D)], semw))

    n = NCHUNK
    fire(0)
    for i in range(1, n):
        if i >= 2:
            writebacks[2 * (i - 2)].wait()
            writebacks[2 * (i - 2) + 1].wait()
        fire(i)
        retire(i - 1)
    retire(n - 1)
    for wb in writebacks[2 * (n - 2):]:
        wb.wait()


def _sc_gather_one(table, idx):
    mesh = plsc.VectorSubcoreMesh(
        core_axis_name="c", subcore_axis_name="s",
        num_cores=NC, num_subcores=NS)
    f = pl.kernel(
        _sc_body,
        out_type=jax.ShapeDtypeStruct((B // 2, 2 * D), jnp.float32),
        mesh=mesh,
        scratch_types=[
            pltpu.VMEM((CHUNK,), jnp.int32),
            pltpu.VMEM((CHUNK,), jnp.int32),
            pltpu.VMEM((CHUNK, D), jnp.float32),
            pltpu.VMEM((CHUNK, D), jnp.float32),
            pltpu.SemaphoreType.DMA,
            pltpu.SemaphoreType.DMA,
        ],
        compiler_params=pltpu.CompilerParams(use_tc_tiling_on_sc=False),
    )
    return f(table, idx)


def _tc_body(g0_ref, g1_ref, g2_ref, g3_ref, v_ref, ind_ref, eid_ref,
             fw_ref, wt_ref, b_ref, pos_ref, ent_ref, o_ref):
    for f, g_ref in enumerate((g0_ref, g1_ref, g2_ref, g3_ref)):
        m = g_ref[...]
        a = jnp.transpose(m[:, :D])
        b = jnp.transpose(m[:, D:])
        o_ref[0, f] = jnp.concatenate([a, b], axis=1)

    v = v_ref[0]
    w = fw_ref[...] * v
    feats = jnp.concatenate([jnp.sin(w), jnp.cos(w)], axis=0)
    proj = lax.dot_general(wt_ref[...], feats, (((0,), (0,)), ((), ())),
                           preferred_element_type=jnp.float32)
    ind = ind_ref[0]
    oh = (lax.broadcasted_iota(jnp.int32, (4, N), 0) == ind
          ).astype(jnp.float32)
    posc = jnp.dot(pos_ref[...], oh,
                   preferred_element_type=jnp.float32)
    o_ref[0, 4] = proj + b_ref[...] + posc

    eid = eid_ref[0]
    ohe = (lax.broadcasted_iota(jnp.int32, (NENT, N), 0) == eid
           ).astype(jnp.float32)
    o_ref[0, 5] = jnp.dot(ent_ref[...], ohe,
                          preferred_element_type=jnp.float32)


def _tc_assemble(gs, values, indicators, ent_idx, linear_w, linear_b,
                 pos_table, ent_table):
    vt = values.T.reshape(L, 1, N)
    it = indicators.T.astype(jnp.int32).reshape(L, 1, N)
    et = ent_idx.T.astype(jnp.int32).reshape(L, 1, N)
    fwc = jnp.asarray(_FW).reshape(PREC, 1)
    wt = linear_w.T
    bias = linear_b.reshape(D, 1)
    post = pos_table.T
    entt = ent_table.T
    out = pl.pallas_call(
        _tc_body,
        grid=(L,),
        in_specs=[
            pl.BlockSpec((H, 128), lambda l: (l, 0)),
            pl.BlockSpec((H, 128), lambda l: (l, 0)),
            pl.BlockSpec((H, 128), lambda l: (l, 0)),
            pl.BlockSpec((H, 128), lambda l: (l, 0)),
            pl.BlockSpec((1, 1, N), lambda l: (l, 0, 0)),
            pl.BlockSpec((1, 1, N), lambda l: (l, 0, 0)),
            pl.BlockSpec((1, 1, N), lambda l: (l, 0, 0)),
            pl.BlockSpec((PREC, 1), lambda l: (0, 0)),
            pl.BlockSpec((2 * PREC, D), lambda l: (0, 0)),
            pl.BlockSpec((D, 1), lambda l: (0, 0)),
            pl.BlockSpec((D, 4), lambda l: (0, 0)),
            pl.BlockSpec((D, NENT), lambda l: (0, 0)),
        ],
        out_specs=pl.BlockSpec((1, C, D, N), lambda l: (l, 0, 0, 0)),
        out_shape=jax.ShapeDtypeStruct((L, C, D, N), jnp.float32),
    )(gs[0], gs[1], gs[2], gs[3], vt, it, et, fwc, wt, bias, post, entt)
    return out


def kernel(disc0_lookup, disc0_table, disc1_lookup, disc1_table,
           disc2_lookup, disc2_table, disc3_lookup, disc3_table,
           cont0_values, cont0_indicators, cont0_linear_w, cont0_linear_b,
           cont0_pos_table, ent0_lookup, ent0_table):
    i0 = disc0_lookup.T.astype(jnp.int32)
    i1 = disc1_lookup.T.astype(jnp.int32)
    i2 = disc2_lookup.T.astype(jnp.int32)
    i3 = disc3_lookup.T.astype(jnp.int32)
    gs = [_sc_gather_one(t, i) for t, i in
          ((disc0_table, i0), (disc1_table, i1),
           (disc2_table, i2), (disc3_table, i3))]
    out = _tc_assemble(gs, cont0_values, cont0_indicators, ent0_lookup,
                       cont0_linear_w, cont0_linear_b, cont0_pos_table,
                       ent0_table)
    return out.transpose(3, 0, 1, 2)


# speedup vs baseline: 4.1818x; 1.1132x over previous
import math

import jax
import jax.numpy as jnp
import numpy as np
from jax import lax
from jax.experimental import pallas as pl
from jax.experimental.pallas import tpu as pltpu
from jax.experimental.pallas import tpu_sc as plsc

N, L, D = 4096, 20, 64
PREC = 8
C = 6
B = N * L
NENT = 204
H = N // 2

NC, NS = 2, 16
NW = NC * NS
BPW = B // NW
CHUNK = 512
SUB = 128
NSUB = CHUNK // SUB
NCHUNK = BPW // CHUNK

_FW = (np.logspace(-PREC, 1, PREC, base=2.0) * math.pi).astype(np.float32)


V = 100004
VB = 1024
NVB = 49
VHALF = NVB * VB
VPAD = 2 * VHALF


def _relayout_body(a_ref, b_ref, o_ref):
    a = jnp.transpose(a_ref[...])
    b = jnp.transpose(b_ref[...])
    o_ref[...] = jnp.concatenate([a, b], axis=1)


def _relayout_table(table):
    tt = table.T
    out = pl.pallas_call(
        _relayout_body,
        grid=(NVB,),
        in_specs=[
            pl.BlockSpec((D, VB), lambda i: (0, i)),
            pl.BlockSpec((D, VB), lambda i: (0, i + NVB)),
        ],
        out_specs=pl.BlockSpec((VB, 2 * D), lambda i: (i, 0)),
        out_shape=jax.ShapeDtypeStruct((VHALF, 2 * D), jnp.float32),
    )(tt, tt)
    return out.reshape(VPAD, D)


HC = CHUNK // 2


def _sc_body(table, idx, out, xa, xb, ra, rb, sem, semw):
    wid = lax.axis_index("s") * NC + lax.axis_index("c")
    base = wid * BPW

    gathers = []
    writebacks = []

    def fire(i):
        xv, rv = (xa, ra) if i % 2 == 0 else (xb, rb)
        off = base + i * CHUNK
        l = off // N
        a0 = pl.multiple_of((off % N) // 2, HC)
        pltpu.sync_copy(idx.at[l, pl.ds(a0, HC)], xv.at[pl.ds(0, HC)])
        pltpu.sync_copy(idx.at[l, pl.ds(H + a0, HC)], xv.at[pl.ds(HC, HC)])
        gs = []
        for j in range(NSUB):
            gs.append(pltpu.async_copy(
                table.at[xv.at[pl.ds(j * SUB, SUB)]],
                rv.at[pl.ds(j * SUB, SUB)], sem))
        gathers.append(gs)

    def retire(i):
        rv = ra if i % 2 == 0 else rb
        off = base + i * CHUNK
        p0 = pl.multiple_of(off // 2, HC)
        for g in gathers[i]:
            g.wait()
        writebacks.append(pltpu.async_copy(
            rv.at[pl.ds(0, HC)],
            out.at[pl.ds(p0, HC), pl.ds(0, D)], semw))
        writebacks.append(pltpu.async_copy(
            rv.at[pl.ds(HC, HC)],
            out.at[pl.ds(p0, HC), pl.ds(D, D)], semw))

    n = NCHUNK
    fire(0)
    for i in range(1, n):
        if i >= 2:
            writebacks[2 * (i - 2)].wait()
            writebacks[2 * (i - 2) + 1].wait()
        fire(i)
        retire(i - 1)
    retire(n - 1)
    for wb in writebacks[2 * (n - 2):]:
        wb.wait()


def _sc_gather_one(table, idx):
    mesh = plsc.VectorSubcoreMesh(
        core_axis_name="c", subcore_axis_name="s",
        num_cores=NC, num_subcores=NS)
    f = pl.kernel(
        _sc_body,
        out_type=jax.ShapeDtypeStruct((B // 2, 2 * D), jnp.float32),
        mesh=mesh,
        scratch_types=[
            pltpu.VMEM((CHUNK,), jnp.int32),
            pltpu.VMEM((CHUNK,), jnp.int32),
            pltpu.VMEM((CHUNK, D), jnp.float32),
            pltpu.VMEM((CHUNK, D), jnp.float32),
            pltpu.SemaphoreType.DMA,
            pltpu.SemaphoreType.DMA,
        ],
        compiler_params=pltpu.CompilerParams(use_tc_tiling_on_sc=False),
    )
    return f(table, idx)


def _tc_body(g0_ref, g1_ref, g2_ref, g3_ref, v_ref, ind_ref, eid_ref,
             fw_ref, wt_ref, b_ref, pos_ref, ent_ref, o_ref):
    for f, g_ref in enumerate((g0_ref, g1_ref, g2_ref, g3_ref)):
        m = g_ref[...]
        a = jnp.transpose(m[:, :D])
        b = jnp.transpose(m[:, D:])
        o_ref[0, f] = jnp.concatenate([a, b], axis=1)

    v = v_ref[0]
    w = fw_ref[...] * v
    feats = jnp.concatenate([jnp.sin(w), jnp.cos(w)], axis=0)
    proj = lax.dot_general(wt_ref[...], feats, (((0,), (0,)), ((), ())),
                           preferred_element_type=jnp.float32)
    ind = ind_ref[0]
    oh = (lax.broadcasted_iota(jnp.int32, (4, N), 0) == ind
          ).astype(jnp.float32)
    posc = jnp.dot(pos_ref[...], oh,
                   preferred_element_type=jnp.float32)
    o_ref[0, 4] = proj + b_ref[...] + posc

    eid = eid_ref[0]
    ohe = (lax.broadcasted_iota(jnp.int32, (NENT, N), 0) == eid
           ).astype(jnp.float32)
    o_ref[0, 5] = jnp.dot(ent_ref[...], ohe,
                          preferred_element_type=jnp.float32)


def _tc_assemble(gs, values, indicators, ent_idx, linear_w, linear_b,
                 pos_table, ent_table):
    vt = values.T.reshape(L, 1, N)
    it = indicators.T.astype(jnp.int32).reshape(L, 1, N)
    et = ent_idx.T.astype(jnp.int32).reshape(L, 1, N)
    fwc = jnp.asarray(_FW).reshape(PREC, 1)
    wt = linear_w.T
    bias = linear_b.reshape(D, 1)
    post = pos_table.T
    entt = ent_table.T
    out = pl.pallas_call(
        _tc_body,
        grid=(L,),
        in_specs=[
            pl.BlockSpec((H, 128), lambda l: (l, 0)),
            pl.BlockSpec((H, 128), lambda l: (l, 0)),
            pl.BlockSpec((H, 128), lambda l: (l, 0)),
            pl.BlockSpec((H, 128), lambda l: (l, 0)),
            pl.BlockSpec((1, 1, N), lambda l: (l, 0, 0)),
            pl.BlockSpec((1, 1, N), lambda l: (l, 0, 0)),
            pl.BlockSpec((1, 1, N), lambda l: (l, 0, 0)),
            pl.BlockSpec((PREC, 1), lambda l: (0, 0)),
            pl.BlockSpec((2 * PREC, D), lambda l: (0, 0)),
            pl.BlockSpec((D, 1), lambda l: (0, 0)),
            pl.BlockSpec((D, 4), lambda l: (0, 0)),
            pl.BlockSpec((D, NENT), lambda l: (0, 0)),
        ],
        out_specs=pl.BlockSpec((1, C, D, N), lambda l: (l, 0, 0, 0)),
        out_shape=jax.ShapeDtypeStruct((L, C, D, N), jnp.float32),
    )(gs[0], gs[1], gs[2], gs[3], vt, it, et, fwc, wt, bias, post, entt)
    return out


def kernel(disc0_lookup, disc0_table, disc1_lookup, disc1_table,
           disc2_lookup, disc2_table, disc3_lookup, disc3_table,
           cont0_values, cont0_indicators, cont0_linear_w, cont0_linear_b,
           cont0_pos_table, ent0_lookup, ent0_table):
    def remap(lookup):
        v = lookup.T.astype(jnp.int32)
        return jnp.where(v < VHALF, v * 2, (v - VHALF) * 2 + 1)

    i0 = remap(disc0_lookup)
    i1 = remap(disc1_lookup)
    i2 = remap(disc2_lookup)
    i3 = remap(disc3_lookup)
    gs = [_sc_gather_one(_relayout_table(t), i) for t, i in
          ((disc0_table, i0), (disc1_table, i1),
           (disc2_table, i2), (disc3_table, i3))]
    out = _tc_assemble(gs, cont0_values, cont0_indicators, ent0_lookup,
                       cont0_linear_w, cont0_linear_b, cont0_pos_table,
                       ent0_table)
    return out.transpose(3, 0, 1, 2)


# speedup vs baseline: 5.2476x; 1.2549x over previous
import math

import jax
import jax.numpy as jnp
import numpy as np
from jax import lax
from jax.experimental import pallas as pl
from jax.experimental.pallas import tpu as pltpu
from jax.experimental.pallas import tpu_sc as plsc

N, L, D = 4096, 20, 64
PREC = 8
C = 6
B = N * L
NENT = 204
H = N // 2

NC, NS = 2, 16
NW = NC * NS
BPW = B // NW
CHUNK = 512
SUB = 128
NSUB = CHUNK // SUB
NCHUNK = BPW // CHUNK

_FW = (np.logspace(-PREC, 1, PREC, base=2.0) * math.pi).astype(np.float32)


V = 100004
VB = 4096
NVB = 13
VHALF = NVB * VB
VPAD = 2 * VHALF


def _relayout_body(a_ref, b_ref, o_ref):
    a = jnp.transpose(a_ref[...])
    b = jnp.transpose(b_ref[...])
    o_ref[...] = jnp.concatenate([a, b], axis=1)


def _relayout_table(table):
    tt = table.T
    out = pl.pallas_call(
        _relayout_body,
        grid=(NVB,),
        in_specs=[
            pl.BlockSpec((D, VB), lambda i: (0, i)),
            pl.BlockSpec(
                (D, VB), lambda i: (0, jnp.minimum(i + NVB, (V - 1) // VB))),
        ],
        out_specs=pl.BlockSpec((VB, 2 * D), lambda i: (i, 0)),
        out_shape=jax.ShapeDtypeStruct((VHALF, 2 * D), jnp.float32),
    )(tt, tt)
    return out.reshape(VPAD, D)


HC = CHUNK // 2


def _sc_body(table, idx, out, xa, xb, ra, rb, sem, semw):
    wid = lax.axis_index("s") * NC + lax.axis_index("c")
    base = wid * BPW

    gathers = []
    writebacks = []

    def fire(i):
        xv, rv = (xa, ra) if i % 2 == 0 else (xb, rb)
        off = base + i * CHUNK
        l = off // N
        a0 = pl.multiple_of((off % N) // 2, HC)
        pltpu.sync_copy(idx.at[l, pl.ds(a0, HC)], xv.at[pl.ds(0, HC)])
        pltpu.sync_copy(idx.at[l, pl.ds(H + a0, HC)], xv.at[pl.ds(HC, HC)])
        gs = []
        for j in range(NSUB):
            gs.append(pltpu.async_copy(
                table.at[xv.at[pl.ds(j * SUB, SUB)]],
                rv.at[pl.ds(j * SUB, SUB)], sem))
        gathers.append(gs)

    def retire(i):
        rv = ra if i % 2 == 0 else rb
        off = base + i * CHUNK
        p0 = pl.multiple_of(off // 2, HC)
        for g in gathers[i]:
            g.wait()
        writebacks.append(pltpu.async_copy(
            rv.at[pl.ds(0, HC)],
            out.at[pl.ds(p0, HC), pl.ds(0, D)], semw))
        writebacks.append(pltpu.async_copy(
            rv.at[pl.ds(HC, HC)],
            out.at[pl.ds(p0, HC), pl.ds(D, D)], semw))

    n = NCHUNK
    fire(0)
    for i in range(1, n):
        if i >= 2:
            writebacks[2 * (i - 2)].wait()
            writebacks[2 * (i - 2) + 1].wait()
        fire(i)
        retire(i - 1)
    retire(n - 1)
    for wb in writebacks[2 * (n - 2):]:
        wb.wait()


def _sc_gather_one(table, idx):
    mesh = plsc.VectorSubcoreMesh(
        core_axis_name="c", subcore_axis_name="s",
        num_cores=NC, num_subcores=NS)
    f = pl.kernel(
        _sc_body,
        out_type=jax.ShapeDtypeStruct((B // 2, 2 * D), jnp.float32),
        mesh=mesh,
        scratch_types=[
            pltpu.VMEM((CHUNK,), jnp.int32),
            pltpu.VMEM((CHUNK,), jnp.int32),
            pltpu.VMEM((CHUNK, D), jnp.float32),
            pltpu.VMEM((CHUNK, D), jnp.float32),
            pltpu.SemaphoreType.DMA,
            pltpu.SemaphoreType.DMA,
        ],
        compiler_params=pltpu.CompilerParams(use_tc_tiling_on_sc=False),
    )
    return f(table, idx)


def _tc_body(g0_ref, g1_ref, g2_ref, g3_ref, v_ref, ind_ref, eid_ref,
             fw_ref, wt_ref, b_ref, pos_ref, ent_ref, o_ref):
    for f, g_ref in enumerate((g0_ref, g1_ref, g2_ref, g3_ref)):
        m = g_ref[...]
        a = jnp.transpose(m[:, :D])
        b = jnp.transpose(m[:, D:])
        o_ref[0, f] = jnp.concatenate([a, b], axis=1)

    v = v_ref[0]
    w = fw_ref[...] * v
    feats = jnp.concatenate([jnp.sin(w), jnp.cos(w)], axis=0)
    proj = lax.dot_general(wt_ref[...], feats, (((0,), (0,)), ((), ())),
                           preferred_element_type=jnp.float32)
    ind = ind_ref[0]
    oh = (lax.broadcasted_iota(jnp.int32, (4, N), 0) == ind
          ).astype(jnp.float32)
    posc = jnp.dot(pos_ref[...], oh,
                   preferred_element_type=jnp.float32)
    o_ref[0, 4] = proj + b_ref[...] + posc

    eid = eid_ref[0]
    ohe = (lax.broadcasted_iota(jnp.int32, (NENT, N), 0) == eid
           ).astype(jnp.float32)
    o_ref[0, 5] = jnp.dot(ent_ref[...], ohe,
                          preferred_element_type=jnp.float32)


def _tc_assemble(gs, values, indicators, ent_idx, linear_w, linear_b,
                 pos_table, ent_table):
    vt = values.T.reshape(L, 1, N)
    it = indicators.T.astype(jnp.int32).reshape(L, 1, N)
    et = ent_idx.T.astype(jnp.int32).reshape(L, 1, N)
    fwc = jnp.asarray(_FW).reshape(PREC, 1)
    wt = linear_w.T
    bias = linear_b.reshape(D, 1)
    post = pos_table.T
    entt = ent_table.T
    out = pl.pallas_call(
        _tc_body,
        grid=(L,),
        in_specs=[
            pl.BlockSpec((H, 128), lambda l: (l, 0)),
            pl.BlockSpec((H, 128), lambda l: (l, 0)),
            pl.BlockSpec((H, 128), lambda l: (l, 0)),
            pl.BlockSpec((H, 128), lambda l: (l, 0)),
            pl.BlockSpec((1, 1, N), lambda l: (l, 0, 0)),
            pl.BlockSpec((1, 1, N), lambda l: (l, 0, 0)),
            pl.BlockSpec((1, 1, N), lambda l: (l, 0, 0)),
            pl.BlockSpec((PREC, 1), lambda l: (0, 0)),
            pl.BlockSpec((2 * PREC, D), lambda l: (0, 0)),
            pl.BlockSpec((D, 1), lambda l: (0, 0)),
            pl.BlockSpec((D, 4), lambda l: (0, 0)),
            pl.BlockSpec((D, NENT), lambda l: (0, 0)),
        ],
        out_specs=pl.BlockSpec((1, C, D, N), lambda l: (l, 0, 0, 0)),
        out_shape=jax.ShapeDtypeStruct((L, C, D, N), jnp.float32),
    )(gs[0], gs[1], gs[2], gs[3], vt, it, et, fwc, wt, bias, post, entt)
    return out


def kernel(disc0_lookup, disc0_table, disc1_lookup, disc1_table,
           disc2_lookup, disc2_table, disc3_lookup, disc3_table,
           cont0_values, cont0_indicators, cont0_linear_w, cont0_linear_b,
           cont0_pos_table, ent0_lookup, ent0_table):
    def remap(lookup):
        v = lookup.T.astype(jnp.int32)
        return jnp.where(v < VHALF, v * 2, (v - VHALF) * 2 + 1)

    i0 = remap(disc0_lookup)
    i1 = remap(disc1_lookup)
    i2 = remap(disc2_lookup)
    i3 = remap(disc3_lookup)
    gs = [_sc_gather_one(_relayout_table(t), i) for t, i in
          ((disc0_table, i0), (disc1_table, i1),
           (disc2_table, i2), (disc3_table, i3))]
    out = _tc_assemble(gs, cont0_values, cont0_indicators, ent0_lookup,
                       cont0_linear_w, cont0_linear_b, cont0_pos_table,
                       ent0_table)
    return out.transpose(3, 0, 1, 2)


# speedup vs baseline: 5.4177x; 1.0324x over previous
import math

import jax
import jax.numpy as jnp
import numpy as np
from jax import lax
from jax.experimental import pallas as pl
from jax.experimental.pallas import tpu as pltpu
from jax.experimental.pallas import tpu_sc as plsc

N, L, D = 4096, 20, 64
PREC = 8
C = 6
B = N * L
NENT = 204
H = N // 2

NC, NS = 2, 16
NW = NC * NS
BPW = B // NW
CHUNK = 512
SUB = 128
NSUB = CHUNK // SUB
NCHUNK = BPW // CHUNK

_FW = (np.logspace(-PREC, 1, PREC, base=2.0) * math.pi).astype(np.float32)


V = 100004
VB = 4096
NVB = 13
VHALF = NVB * VB
VPAD = 2 * VHALF


def _relayout_body(a_ref, b_ref, eye_ref, o_ref):
    eye = eye_ref[...]
    dn = (((0,), (0,)), ((), ()))
    a = lax.dot_general(a_ref[...], eye, dn,
                        preferred_element_type=jnp.float32)
    b = lax.dot_general(b_ref[...], eye, dn,
                        preferred_element_type=jnp.float32)
    o_ref[...] = jnp.concatenate([a, b], axis=1)


def _relayout_table(table, eye):
    tt = table.T
    out = pl.pallas_call(
        _relayout_body,
        grid=(NVB,),
        in_specs=[
            pl.BlockSpec((D, VB), lambda i: (0, i)),
            pl.BlockSpec(
                (D, VB), lambda i: (0, jnp.minimum(i + NVB, (V - 1) // VB))),
            pl.BlockSpec((D, D), lambda i: (0, 0)),
        ],
        out_specs=pl.BlockSpec((VB, 2 * D), lambda i: (i, 0)),
        out_shape=jax.ShapeDtypeStruct((VHALF, 2 * D), jnp.float32),
    )(tt, tt, eye)
    return out.reshape(VPAD, D)


HC = CHUNK // 2


def _sc_body(table, idx, out, xa, xb, ra, rb, sem, semw):
    wid = lax.axis_index("s") * NC + lax.axis_index("c")
    base = wid * BPW

    gathers = []
    writebacks = []

    def fire(i):
        xv, rv = (xa, ra) if i % 2 == 0 else (xb, rb)
        off = base + i * CHUNK
        l = off // N
        a0 = pl.multiple_of((off % N) // 2, HC)
        pltpu.sync_copy(idx.at[l, pl.ds(a0, HC)], xv.at[pl.ds(0, HC)])
        pltpu.sync_copy(idx.at[l, pl.ds(H + a0, HC)], xv.at[pl.ds(HC, HC)])
        gs = []
        for j in range(NSUB):
            gs.append(pltpu.async_copy(
                table.at[xv.at[pl.ds(j * SUB, SUB)]],
                rv.at[pl.ds(j * SUB, SUB)], sem))
        gathers.append(gs)

    def retire(i):
        rv = ra if i % 2 == 0 else rb
        off = base + i * CHUNK
        p0 = pl.multiple_of(off // 2, HC)
        for g in gathers[i]:
            g.wait()
        writebacks.append(pltpu.async_copy(
            rv.at[pl.ds(0, HC)],
            out.at[pl.ds(p0, HC), pl.ds(0, D)], semw))
        writebacks.append(pltpu.async_copy(
            rv.at[pl.ds(HC, HC)],
            out.at[pl.ds(p0, HC), pl.ds(D, D)], semw))

    n = NCHUNK
    fire(0)
    for i in range(1, n):
        if i >= 2:
            writebacks[2 * (i - 2)].wait()
            writebacks[2 * (i - 2) + 1].wait()
        fire(i)
        retire(i - 1)
    retire(n - 1)
    for wb in writebacks[2 * (n - 2):]:
        wb.wait()


def _sc_gather_one(table, idx):
    mesh = plsc.VectorSubcoreMesh(
        core_axis_name="c", subcore_axis_name="s",
        num_cores=NC, num_subcores=NS)
    f = pl.kernel(
        _sc_body,
        out_type=jax.ShapeDtypeStruct((B // 2, 2 * D), jnp.float32),
        mesh=mesh,
        scratch_types=[
            pltpu.VMEM((CHUNK,), jnp.int32),
            pltpu.VMEM((CHUNK,), jnp.int32),
            pltpu.VMEM((CHUNK, D), jnp.float32),
            pltpu.VMEM((CHUNK, D), jnp.float32),
            pltpu.SemaphoreType.DMA,
            pltpu.SemaphoreType.DMA,
        ],
        compiler_params=pltpu.CompilerParams(use_tc_tiling_on_sc=False),
    )
    return f(table, idx)


def _tc_body(g0_ref, g1_ref, g2_ref, g3_ref, v_ref, ind_ref, eid_ref,
             fw_ref, wt_ref, b_ref, pos_ref, ent_ref, eye_ref, o_ref):
    eye = eye_ref[...]
    dn = (((0,), (1,)), ((), ()))
    for f, g_ref in enumerate((g0_ref, g1_ref, g2_ref, g3_ref)):
        m = g_ref[...]
        a = lax.dot_general(eye, m[:, :D], dn,
                            preferred_element_type=jnp.float32)
        b = lax.dot_general(eye, m[:, D:], dn,
                            preferred_element_type=jnp.float32)
        o_ref[0, f] = jnp.concatenate([a, b], axis=1)

    v = v_ref[0]
    w = fw_ref[...] * v
    feats = jnp.concatenate([jnp.sin(w), jnp.cos(w)], axis=0)
    proj = lax.dot_general(wt_ref[...], feats, (((0,), (0,)), ((), ())),
                           preferred_element_type=jnp.float32)
    ind = ind_ref[0]
    oh = (lax.broadcasted_iota(jnp.int32, (4, N), 0) == ind
          ).astype(jnp.float32)
    posc = jnp.dot(pos_ref[...], oh,
                   preferred_element_type=jnp.float32)
    o_ref[0, 4] = proj + b_ref[...] + posc

    eid = eid_ref[0]
    ohe = (lax.broadcasted_iota(jnp.int32, (NENT, N), 0) == eid
           ).astype(jnp.float32)
    o_ref[0, 5] = jnp.dot(ent_ref[...], ohe,
                          preferred_element_type=jnp.float32)


def _tc_assemble(gs, values, indicators, ent_idx, linear_w, linear_b,
                 pos_table, ent_table):
    vt = values.T.reshape(L, 1, N)
    it = indicators.T.astype(jnp.int32).reshape(L, 1, N)
    et = ent_idx.T.astype(jnp.int32).reshape(L, 1, N)
    fwc = jnp.asarray(_FW).reshape(PREC, 1)
    wt = linear_w.T
    bias = linear_b.reshape(D, 1)
    post = pos_table.T
    entt = ent_table.T
    out = pl.pallas_call(
        _tc_body,
        grid=(L,),
        in_specs=[
            pl.BlockSpec((H, 128), lambda l: (l, 0)),
            pl.BlockSpec((H, 128), lambda l: (l, 0)),
            pl.BlockSpec((H, 128), lambda l: (l, 0)),
            pl.BlockSpec((H, 128), lambda l: (l, 0)),
            pl.BlockSpec((1, 1, N), lambda l: (l, 0, 0)),
            pl.BlockSpec((1, 1, N), lambda l: (l, 0, 0)),
            pl.BlockSpec((1, 1, N), lambda l: (l, 0, 0)),
            pl.BlockSpec((PREC, 1), lambda l: (0, 0)),
            pl.BlockSpec((2 * PREC, D), lambda l: (0, 0)),
            pl.BlockSpec((D, 1), lambda l: (0, 0)),
            pl.BlockSpec((D, 4), lambda l: (0, 0)),
            pl.BlockSpec((D, NENT), lambda l: (0, 0)),
            pl.BlockSpec((D, D), lambda l: (0, 0)),
        ],
        out_specs=pl.BlockSpec((1, C, D, N), lambda l: (l, 0, 0, 0)),
        out_shape=jax.ShapeDtypeStruct((L, C, D, N), jnp.float32),
    )(gs[0], gs[1], gs[2], gs[3], vt, it, et, fwc, wt, bias, post, entt,
      jnp.eye(D, dtype=jnp.float32))
    return out


def kernel(disc0_lookup, disc0_table, disc1_lookup, disc1_table,
           disc2_lookup, disc2_table, disc3_lookup, disc3_table,
           cont0_values, cont0_indicators, cont0_linear_w, cont0_linear_b,
           cont0_pos_table, ent0_lookup, ent0_table):
    def remap(lookup):
        v = lookup.T.astype(jnp.int32)
        return jnp.where(v < VHALF, v * 2, (v - VHALF) * 2 + 1)

    i0 = remap(disc0_lookup)
    i1 = remap(disc1_lookup)
    i2 = remap(disc2_lookup)
    i3 = remap(disc3_lookup)
    eye = jnp.eye(D, dtype=jnp.float32)
    gs = [_sc_gather_one(_relayout_table(t, eye), i) for t, i in
          ((disc0_table, i0), (disc1_table, i1),
           (disc2_table, i2), (disc3_table, i3))]
    out = _tc_assemble(gs, cont0_values, cont0_indicators, ent0_lookup,
                       cont0_linear_w, cont0_linear_b, cont0_pos_table,
                       ent0_table)
    return out.transpose(3, 0, 1, 2)


# speedup vs baseline: 5.5106x; 1.0171x over previous
import math

import jax
import jax.numpy as jnp
import numpy as np
from jax import lax
from jax.experimental import pallas as pl
from jax.experimental.pallas import tpu as pltpu
from jax.experimental.pallas import tpu_sc as plsc

N, L, D = 4096, 20, 64
PREC = 8
C = 6
B = N * L
NENT = 204
H = N // 2

NC, NS = 2, 16
NW = NC * NS
BPW = B // NW
CHUNK = 512
SUB = 128
NSUB = CHUNK // SUB
NCHUNK = BPW // CHUNK

_FW = (np.logspace(-PREC, 1, PREC, base=2.0) * math.pi).astype(np.float32)


V = 100004
VB = 8192
NVB = 7
VHALF = NVB * VB
VPAD = 2 * VHALF


def _relayout_body(a_ref, b_ref, eye_ref, o_ref):
    eye = eye_ref[...]
    dn = (((0,), (0,)), ((), ()))
    a = lax.dot_general(a_ref[...], eye, dn,
                        preferred_element_type=jnp.float32)
    b = lax.dot_general(b_ref[...], eye, dn,
                        preferred_element_type=jnp.float32)
    o_ref[...] = jnp.concatenate([a, b], axis=1)


def _relayout_table(table, eye):
    tt = table.T
    out = pl.pallas_call(
        _relayout_body,
        grid=(NVB,),
        in_specs=[
            pl.BlockSpec((D, VB), lambda i: (0, i)),
            pl.BlockSpec(
                (D, VB), lambda i: (0, jnp.minimum(i + NVB, (V - 1) // VB))),
            pl.BlockSpec((D, D), lambda i: (0, 0)),
        ],
        out_specs=pl.BlockSpec((VB, 2 * D), lambda i: (i, 0)),
        out_shape=jax.ShapeDtypeStruct((VHALF, 2 * D), jnp.float32),
    )(tt, tt, eye)
    return out.reshape(VPAD, D)


HC = CHUNK // 2


def _sc_body(table, idx, out, xa, xb, ra, rb, sem, semw):
    wid = lax.axis_index("s") * NC + lax.axis_index("c")
    base = wid * BPW

    gathers = []
    writebacks = []

    def fire(i):
        xv, rv = (xa, ra) if i % 2 == 0 else (xb, rb)
        off = base + i * CHUNK
        l = off // N
        a0 = pl.multiple_of((off % N) // 2, HC)
        pltpu.sync_copy(idx.at[l, pl.ds(a0, HC)], xv.at[pl.ds(0, HC)])
        pltpu.sync_copy(idx.at[l, pl.ds(H + a0, HC)], xv.at[pl.ds(HC, HC)])
        gs = []
        for j in range(NSUB):
            gs.append(pltpu.async_copy(
                table.at[xv.at[pl.ds(j * SUB, SUB)]],
                rv.at[pl.ds(j * SUB, SUB)], sem))
        gathers.append(gs)

    def retire(i):
        rv = ra if i % 2 == 0 else rb
        off = base + i * CHUNK
        p0 = pl.multiple_of(off // 2, HC)
        for g in gathers[i]:
            g.wait()
        writebacks.append(pltpu.async_copy(
            rv.at[pl.ds(0, HC)],
            out.at[pl.ds(p0, HC), pl.ds(0, D)], semw))
        writebacks.append(pltpu.async_copy(
            rv.at[pl.ds(HC, HC)],
            out.at[pl.ds(p0, HC), pl.ds(D, D)], semw))

    n = NCHUNK
    fire(0)
    for i in range(1, n):
        if i >= 2:
            writebacks[2 * (i - 2)].wait()
            writebacks[2 * (i - 2) + 1].wait()
        fire(i)
        retire(i - 1)
    retire(n - 1)
    for wb in writebacks[2 * (n - 2):]:
        wb.wait()


def _sc_gather_one(table, idx):
    mesh = plsc.VectorSubcoreMesh(
        core_axis_name="c", subcore_axis_name="s",
        num_cores=NC, num_subcores=NS)
    f = pl.kernel(
        _sc_body,
        out_type=jax.ShapeDtypeStruct((B // 2, 2 * D), jnp.float32),
        mesh=mesh,
        scratch_types=[
            pltpu.VMEM((CHUNK,), jnp.int32),
            pltpu.VMEM((CHUNK,), jnp.int32),
            pltpu.VMEM((CHUNK, D), jnp.float32),
            pltpu.VMEM((CHUNK, D), jnp.float32),
            pltpu.SemaphoreType.DMA,
            pltpu.SemaphoreType.DMA,
        ],
        compiler_params=pltpu.CompilerParams(use_tc_tiling_on_sc=False),
    )
    return f(table, idx)


def _tc_body(g0_ref, g1_ref, g2_ref, g3_ref, v_ref, ind_ref, eid_ref,
             fw_ref, wt_ref, b_ref, pos_ref, ent_ref, eye_ref, o_ref):
    eye = eye_ref[...]
    dn = (((0,), (1,)), ((), ()))
    for f, g_ref in enumerate((g0_ref, g1_ref, g2_ref, g3_ref)):
        m = g_ref[...]
        a = lax.dot_general(eye, m[:, :D], dn,
                            preferred_element_type=jnp.float32)
        b = lax.dot_general(eye, m[:, D:], dn,
                            preferred_element_type=jnp.float32)
        o_ref[0, f] = jnp.concatenate([a, b], axis=1)

    v = v_ref[0]
    w = fw_ref[...] * v
    feats = jnp.concatenate([jnp.sin(w), jnp.cos(w)], axis=0)
    proj = lax.dot_general(wt_ref[...], feats, (((0,), (0,)), ((), ())),
                           preferred_element_type=jnp.float32)
    ind = ind_ref[0]
    oh = (lax.broadcasted_iota(jnp.int32, (4, N), 0) == ind
          ).astype(jnp.float32)
    posc = jnp.dot(pos_ref[...], oh,
                   preferred_element_type=jnp.float32)
    o_ref[0, 4] = proj + b_ref[...] + posc

    eid = eid_ref[0]
    ohe = (lax.broadcasted_iota(jnp.int32, (NENT, N), 0) == eid
           ).astype(jnp.float32)
    o_ref[0, 5] = jnp.dot(ent_ref[...], ohe,
                          preferred_element_type=jnp.float32)


def _tc_assemble(gs, values, indicators, ent_idx, linear_w, linear_b,
                 pos_table, ent_table):
    vt = values.T.reshape(L, 1, N)
    it = indicators.T.astype(jnp.int32).reshape(L, 1, N)
    et = ent_idx.T.astype(jnp.int32).reshape(L, 1, N)
    fwc = jnp.asarray(_FW).reshape(PREC, 1)
    wt = linear_w.T
    bias = linear_b.reshape(D, 1)
    post = pos_table.T
    entt = ent_table.T
    out = pl.pallas_call(
        _tc_body,
        grid=(L,),
        in_specs=[
            pl.BlockSpec((H, 128), lambda l: (l, 0)),
            pl.BlockSpec((H, 128), lambda l: (l, 0)),
            pl.BlockSpec((H, 128), lambda l: (l, 0)),
            pl.BlockSpec((H, 128), lambda l: (l, 0)),
            pl.BlockSpec((1, 1, N), lambda l: (l, 0, 0)),
            pl.BlockSpec((1, 1, N), lambda l: (l, 0, 0)),
            pl.BlockSpec((1, 1, N), lambda l: (l, 0, 0)),
            pl.BlockSpec((PREC, 1), lambda l: (0, 0)),
            pl.BlockSpec((2 * PREC, D), lambda l: (0, 0)),
            pl.BlockSpec((D, 1), lambda l: (0, 0)),
            pl.BlockSpec((D, 4), lambda l: (0, 0)),
            pl.BlockSpec((D, NENT), lambda l: (0, 0)),
            pl.BlockSpec((D, D), lambda l: (0, 0)),
        ],
        out_specs=pl.BlockSpec((1, C, D, N), lambda l: (l, 0, 0, 0)),
        out_shape=jax.ShapeDtypeStruct((L, C, D, N), jnp.float32),
    )(gs[0], gs[1], gs[2], gs[3], vt, it, et, fwc, wt, bias, post, entt,
      jnp.eye(D, dtype=jnp.float32))
    return out


def kernel(disc0_lookup, disc0_table, disc1_lookup, disc1_table,
           disc2_lookup, disc2_table, disc3_lookup, disc3_table,
           cont0_values, cont0_indicators, cont0_linear_w, cont0_linear_b,
           cont0_pos_table, ent0_lookup, ent0_table):
    def remap(lookup):
        v = lookup.T.astype(jnp.int32)
        return jnp.where(v < VHALF, v * 2, (v - VHALF) * 2 + 1)

    i0 = remap(disc0_lookup)
    i1 = remap(disc1_lookup)
    i2 = remap(disc2_lookup)
    i3 = remap(disc3_lookup)
    eye = jnp.eye(D, dtype=jnp.float32)
    gs = [_sc_gather_one(_relayout_table(t, eye), i) for t, i in
          ((disc0_table, i0), (disc1_table, i1),
           (disc2_table, i2), (disc3_table, i3))]
    out = _tc_assemble(gs, cont0_values, cont0_indicators, ent0_lookup,
                       cont0_linear_w, cont0_linear_b, cont0_pos_table,
                       ent0_table)
    return out.transpose(3, 0, 1, 2)
